# baseline (speedup 1.0000x reference)
import jax
import jax.numpy as jnp
from jax import lax
from jax.experimental import pallas as pl
from jax.experimental.pallas import tpu as pltpu
from jax.experimental.pallas import tpu_sc as plsc

_B = 4096
_E = 32
_NF = 8

_info = plsc.get_sparse_core_info()
_NC, _NS = _info.num_cores, _info.num_subcores
_NW = _NC * _NS
_BPW = _B // _NW


def _sc_gather_body(*refs):
    sid_refs = refs[0:_NF]
    tab_refs = refs[_NF:2 * _NF]
    out_refs = refs[2 * _NF:3 * _NF]
    sidx_v, slab_v, sem_g, sem_w = refs[3 * _NF:]

    wid = lax.axis_index("s") * _NC + lax.axis_index("c")
    base = wid * _BPW

    for f in range(_NF):
        pltpu.sync_copy(sid_refs[f].at[pl.ds(base, _BPW)], sidx_v.at[f])

    def gather(f):
        return pltpu.async_copy(
            tab_refs[f].at[sidx_v.at[f]], slab_v.at[f & 1], sem_g)

    cp = [gather(0), None]
    wr = [None, None]
    for f in range(_NF):
        b = f & 1
        cp[b].wait()
        if f + 1 < _NF:
            if wr[1 - b] is not None:
                wr[1 - b].wait()
            cp[1 - b] = gather(f + 1)
        wr[b] = pltpu.async_copy(
            slab_v.at[b], out_refs[f].at[pl.ds(base, _BPW), :], sem_w)
    wr[0].wait()
    wr[1].wait()


_sc_gather = pl.kernel(
    _sc_gather_body,
    out_type=[jax.ShapeDtypeStruct((_B, 4 * _E), jnp.float32)] * _NF,
    mesh=plsc.VectorSubcoreMesh(core_axis_name="c", subcore_axis_name="s"),
    scratch_types=[
        pltpu.VMEM((_NF, _BPW), jnp.int32),
        pltpu.VMEM((2, _BPW, 4 * _E), jnp.float32),
        pltpu.SemaphoreType.DMA,
        pltpu.SemaphoreType.DMA,
    ],
)


_BM = 1024
_NBLK = _B // _BM


def _tower_body(*refs):
    slabs = refs[0:_NF]
    rems = refs[_NF:2 * _NF]
    (uw1, ub1, uw2, ub2, iw1, ib1, iw2, ib2) = refs[2 * _NF:3 * _NF]
    out = refs[3 * _NF]
    acc = refs[3 * _NF + 1]

    def pick(f):
        s = slabs[f][...]
        r = rems[f][...]
        x = jnp.where(r == 0, s[:, 0:32], 0.0)
        for g in range(1, 4):
            x = x + jnp.where(r == g, s[:, 32 * g:32 * g + 32], 0.0)
        return x

    xu = jnp.concatenate([pick(f) for f in range(4)], axis=1)
    xi = jnp.concatenate([pick(f) for f in range(4, 8)], axis=1)

    u = jnp.maximum(
        jnp.dot(xu, uw1[...], preferred_element_type=jnp.float32)
        + ub1[...], 0.0)
    u = jnp.maximum(
        jnp.dot(u, uw2[...], preferred_element_type=jnp.float32)
        + ub2[...], 0.0)
    v = jnp.maximum(
        jnp.dot(xi, iw1[...], preferred_element_type=jnp.float32)
        + ib1[...], 0.0)
    v = jnp.maximum(
        jnp.dot(v, iw2[...], preferred_element_type=jnp.float32)
        + ib2[...], 0.0)
    s_ui = jnp.sum(u * v)
    s_uu = jnp.sum(u * u)
    s_ii = jnp.sum(v * v)

    step = pl.program_id(0)

    @pl.when(step == 0)
    def _init():
        acc[0] = s_ui
        acc[1] = s_uu
        acc[2] = s_ii

    @pl.when(step > 0)
    def _accum():
        acc[0] += s_ui
        acc[1] += s_uu
        acc[2] += s_ii

    @pl.when(step == _NBLK - 1)
    def _final():
        out[0, 0] = acc[0] / jnp.sqrt(acc[1] * acc[2])


_tower = pl.pallas_call(
    _tower_body,
    grid=(_NBLK,),
    out_shape=jax.ShapeDtypeStruct((1, 1), jnp.float32),
    in_specs=(
        [pl.BlockSpec((_BM, 4 * _E), lambda i: (i, 0))] * _NF
        + [pl.BlockSpec((_BM, 1), lambda i: (i, 0))] * _NF
        + [pl.BlockSpec(s, lambda i: (0, 0))
           for s in ((128, 64), (1, 64), (64, 32), (1, 32)) * 2]
    ),
    out_specs=pl.BlockSpec(memory_space=pltpu.SMEM),
    scratch_shapes=[pltpu.SMEM((3,), jnp.float32)],
)


def kernel(idx_u0, idx_u1, idx_u2, idx_u3, idx_i0, idx_i1, idx_i2, idx_i3,
           table_u0, table_u1, table_u2, table_u3,
           table_i0, table_i1, table_i2, table_i3,
           user_W1, user_b1, user_W2, user_b2,
           item_W1, item_b1, item_W2, item_b2):
    idxs = [x.reshape(-1).astype(jnp.int32)
            for x in (idx_u0, idx_u1, idx_u2, idx_u3,
                      idx_i0, idx_i1, idx_i2, idx_i3)]
    sids = [x >> 2 for x in idxs]
    rems = [(x & 3).reshape(_B, 1) for x in idxs]
    tabs = [t.reshape(t.shape[0] // 4, 4 * _E)
            for t in (table_u0, table_u1, table_u2, table_u3,
                      table_i0, table_i1, table_i2, table_i3)]
    slabs = _sc_gather(*sids, *tabs)
    out = _tower(*slabs, *rems,
                 user_W1, user_b1.reshape(1, 64), user_W2,
                 user_b2.reshape(1, 32),
                 item_W1, item_b1.reshape(1, 64), item_W2,
                 item_b2.reshape(1, 32))
    return out[0, 0]


# speedup vs baseline: 3.8326x; 3.8326x over previous
import jax
import jax.numpy as jnp
from jax import lax
from jax.experimental import pallas as pl
from jax.experimental.pallas import tpu as pltpu
from jax.experimental.pallas import tpu_sc as plsc

_B = 4096
_E = 32
_NF = 8
_NAT = (0, 4)
_SLB = (1, 2, 3, 5, 6, 7)

_info = plsc.get_sparse_core_info()
_NC, _NS = _info.num_cores, _info.num_subcores
_NW = _NC * _NS
_BPW = _B // _NW
_SUB = 8


def _sc_gather_body(*refs):
    sid_refs = refs[0:_NF]
    nat_refs = {f: refs[_NF + i] for i, f in enumerate(_NAT)}
    tab_refs = {f: refs[_NF + 2 + i] for i, f in enumerate(_SLB)}
    out_refs = refs[_NF + 8:_NF + 16]
    sidx_v, slab_v, stack_v, sem_g, sem_w, sem_n = refs[_NF + 16:]

    wid = lax.axis_index("s") * _NC + lax.axis_index("c")
    base = wid * _BPW

    for f in range(_NF):
        pltpu.sync_copy(sid_refs[f].at[pl.ds(base, _BPW)], sidx_v.at[f])

    iota = lax.iota(jnp.int32, 16)
    pair = lax.shift_right_logical(iota, 1)

    for nf, f in enumerate(_NAT):
        tab = nat_refs[f]
        pbuf = slab_v.at[nf]
        fsplat = jnp.full((16,), f, jnp.int32)

        @pl.loop(0, _BPW // 16)
        def _chunk(k):
            ids = sidx_v[f, pl.ds(k * 16, 16)]

            def fire(sub):
                cps = []
                for l in range(_SUB):
                    j = ids[sub * _SUB + l]
                    off = pl.multiple_of(
                        lax.shift_left(
                            lax.shift_right_logical(j, 7), 7), 128)
                    cps.append(pltpu.async_copy(
                        tab.at[:, pl.ds(off, 128)],
                        stack_v.at[sub, l], sem_n))
                return cps

            def extract(sub):
                row0 = k * 16 + sub * _SUB
                rows = jnp.full((16,), row0, jnp.int32) + pair
                idv = plsc.load_gather(sidx_v, [fsplat, rows])
                cv = idv & jnp.full((16,), 127, jnp.int32)
                colbase = lax.shift_left(
                    rows & jnp.full((16,), 3, jnp.int32), 5)
                ebit = iota & jnp.full((16,), 1, jnp.int32)
                for e0 in range(0, _E, 2):
                    esplat = jnp.full((16,), e0, jnp.int32) + ebit
                    vals = plsc.load_gather(
                        stack_v.at[sub], [pair, esplat, cv])
                    plsc.store_scatter(pbuf, [rows, colbase + esplat], vals)

            c0 = fire(0)
            c1 = fire(1)
            for cp in c0:
                cp.wait()
            extract(0)
            for cp in c1:
                cp.wait()
            extract(1)

        pltpu.sync_copy(pbuf, out_refs[f].at[pl.ds(base, _BPW), :])

    def gather(k):
        f = _SLB[k]
        return pltpu.async_copy(
            tab_refs[f].at[sidx_v.at[f]], slab_v.at[k & 1], sem_g)

    cp = [gather(0), None]
    wr = [None, None]
    for k in range(len(_SLB)):
        b = k & 1
        cp[b].wait()
        if k + 1 < len(_SLB):
            if wr[1 - b] is not None:
                wr[1 - b].wait()
            cp[1 - b] = gather(k + 1)
        wr[b] = pltpu.async_copy(
            slab_v.at[b], out_refs[_SLB[k]].at[pl.ds(base, _BPW), :], sem_w)
    wr[0].wait()
    wr[1].wait()


_sc_gather = pl.kernel(
    _sc_gather_body,
    out_type=[jax.ShapeDtypeStruct((_B, 4 * _E), jnp.float32)] * _NF,
    mesh=plsc.VectorSubcoreMesh(core_axis_name="c", subcore_axis_name="s"),
    compiler_params=pltpu.CompilerParams(needs_layout_passes=False),
    scratch_types=[
        pltpu.VMEM((_NF, _BPW), jnp.int32),
        pltpu.VMEM((2, _BPW, 4 * _E), jnp.float32),
        pltpu.VMEM((2, _SUB, _E, 128), jnp.float32),
        pltpu.SemaphoreType.DMA,
        pltpu.SemaphoreType.DMA,
        pltpu.SemaphoreType.DMA,
    ],
)


_BM = 1024
_NBLK = _B // _BM


def _tower_body(*refs):
    slabs = refs[0:_NF]
    rems = refs[_NF:2 * _NF]
    (uw1, ub1, uw2, ub2, iw1, ib1, iw2, ib2) = refs[2 * _NF:3 * _NF]
    out = refs[3 * _NF]
    acc = refs[3 * _NF + 1]

    def pick(f):
        s = slabs[f][...]
        r = rems[f][...]
        x = jnp.where(r == 0, s[:, 0:_E], 0.0)
        for g in range(1, 4):
            x = x + jnp.where(r == g, s[:, _E * g:_E * g + _E], 0.0)
        return x

    xu = jnp.concatenate([pick(f) for f in range(4)], axis=1)
    xi = jnp.concatenate([pick(f) for f in range(4, 8)], axis=1)

    u = jnp.maximum(
        jnp.dot(xu, uw1[...], preferred_element_type=jnp.float32)
        + ub1[...], 0.0)
    u = jnp.maximum(
        jnp.dot(u, uw2[...], preferred_element_type=jnp.float32)
        + ub2[...], 0.0)
    v = jnp.maximum(
        jnp.dot(xi, iw1[...], preferred_element_type=jnp.float32)
        + ib1[...], 0.0)
    v = jnp.maximum(
        jnp.dot(v, iw2[...], preferred_element_type=jnp.float32)
        + ib2[...], 0.0)
    s_ui = jnp.sum(u * v)
    s_uu = jnp.sum(u * u)
    s_ii = jnp.sum(v * v)

    step = pl.program_id(0)

    @pl.when(step == 0)
    def _init():
        acc[0] = s_ui
        acc[1] = s_uu
        acc[2] = s_ii

    @pl.when(step > 0)
    def _accum():
        acc[0] += s_ui
        acc[1] += s_uu
        acc[2] += s_ii

    @pl.when(step == _NBLK - 1)
    def _final():
        out[0, 0] = acc[0] / jnp.sqrt(acc[1] * acc[2])


_tower = pl.pallas_call(
    _tower_body,
    grid=(_NBLK,),
    out_shape=jax.ShapeDtypeStruct((1, 1), jnp.float32),
    in_specs=(
        [pl.BlockSpec((_BM, 4 * _E), lambda i: (i, 0))] * _NF
        + [pl.BlockSpec((_BM, 1), lambda i: (i, 0))] * _NF
        + [pl.BlockSpec(s, lambda i: (0, 0))
           for s in ((128, 64), (1, 64), (64, 32), (1, 32)) * 2]
    ),
    out_specs=pl.BlockSpec(memory_space=pltpu.SMEM),
    scratch_shapes=[pltpu.SMEM((3,), jnp.float32)],
)


def kernel(idx_u0, idx_u1, idx_u2, idx_u3, idx_i0, idx_i1, idx_i2, idx_i3,
           table_u0, table_u1, table_u2, table_u3,
           table_i0, table_i1, table_i2, table_i3,
           user_W1, user_b1, user_W2, user_b2,
           item_W1, item_b1, item_W2, item_b2):
    idxs = [x.reshape(-1).astype(jnp.int32)
            for x in (idx_u0, idx_u1, idx_u2, idx_u3,
                      idx_i0, idx_i1, idx_i2, idx_i3)]
    tables = (table_u0, table_u1, table_u2, table_u3,
              table_i0, table_i1, table_i2, table_i3)
    sids = [idxs[f] if f in _NAT else idxs[f] >> 2 for f in range(_NF)]
    bmod4 = (jnp.arange(_B, dtype=jnp.int32) & 3).reshape(_B, 1)
    rems = [bmod4 if f in _NAT else (idxs[f] & 3).reshape(_B, 1)
            for f in range(_NF)]
    nats = [tables[f].T for f in _NAT]
    slabs = [tables[f].reshape(tables[f].shape[0] // 4, 4 * _E) for f in _SLB]
    outs = _sc_gather(*sids, *nats, *slabs)
    out = _tower(*outs, *rems,
                 user_W1, user_b1.reshape(1, 64), user_W2,
                 user_b2.reshape(1, 32),
                 item_W1, item_b1.reshape(1, 64), item_W2,
                 item_b2.reshape(1, 32))
    return out[0, 0]
